# baseline (speedup 1.0000x reference)
import functools

import jax
import jax.numpy as jnp
from jax import lax
from jax.experimental import pallas as pl
from jax.experimental.pallas import tpu as pltpu
from jax.experimental.pallas import tpu_sc as plsc

CHUNK = 128


def _build_lookup(n_total, emb):
    info = plsc.get_sparse_core_info()
    nc, ns = info.num_cores, info.num_subcores
    nw = nc * ns
    per_w = n_total // nw
    n_chunks = per_w // CHUNK

    mesh = plsc.VectorSubcoreMesh(core_axis_name="c", subcore_axis_name="s")

    @functools.partial(
        pl.kernel,
        mesh=mesh,
        out_type=jax.ShapeDtypeStruct((n_total, emb), jnp.float32),
        scratch_types=[
            pltpu.VMEM((CHUNK,), jnp.int32),
            pltpu.VMEM((CHUNK, emb), jnp.float32),
            pltpu.SemaphoreType.DMA,
        ],
        compiler_params=pltpu.CompilerParams(use_tc_tiling_on_sc=False),
    )
    def lookup(idx_hbm, table_hbm, out_hbm, idx_v, rows_v, sem):
        wid = lax.axis_index("s") * nc + lax.axis_index("c")
        base = wid * per_w

        def body(j, carry):
            off = base + j * CHUNK
            pltpu.sync_copy(idx_hbm.at[pl.ds(off, CHUNK)], idx_v)
            pltpu.async_copy(table_hbm.at[idx_v], rows_v, sem).wait()
            pltpu.sync_copy(rows_v, out_hbm.at[pl.ds(off, CHUNK)])
            return carry

        lax.fori_loop(0, n_chunks, body, 0)

    return lookup


def kernel(x, table):
    b, l = x.shape
    emb = table.shape[1]
    n_total = b * l
    xf = x.reshape(n_total)
    out = _build_lookup(n_total, emb)(xf, table)
    return out.reshape(b, l, emb)


# speedup vs baseline: 1.1982x; 1.1982x over previous
import functools

import jax
import jax.numpy as jnp
from jax import lax
from jax.experimental import pallas as pl
from jax.experimental.pallas import tpu as pltpu
from jax.experimental.pallas import tpu_sc as plsc

CHUNK = 128
GROUP = 4
NBUF = 2
ROWS = CHUNK * GROUP


def _build_lookup(n_total, emb):
    info = plsc.get_sparse_core_info()
    nc, ns = info.num_cores, info.num_subcores
    nw = nc * ns
    per_w = n_total // nw
    idx_rows_w = per_w // CHUNK
    n_groups = per_w // ROWS

    mesh = plsc.VectorSubcoreMesh(core_axis_name="c", subcore_axis_name="s")

    @functools.partial(
        pl.kernel,
        mesh=mesh,
        out_type=jax.ShapeDtypeStruct((n_total, emb), jnp.float32),
        scratch_types=[
            pltpu.VMEM((idx_rows_w, CHUNK), jnp.int32),
            pltpu.VMEM((ROWS, emb), jnp.float32),
            pltpu.VMEM((ROWS, emb), jnp.float32),
            pltpu.SemaphoreType.DMA,
            pltpu.SemaphoreType.DMA,
        ],
        compiler_params=pltpu.CompilerParams(use_tc_tiling_on_sc=False),
    )
    def lookup(idx_hbm, table_hbm, out_hbm, idx_v, buf0, buf1, sem0, sem1):
        wid = lax.axis_index("s") * nc + lax.axis_index("c")
        base = wid * per_w
        bufs = (buf0, buf1)
        sems = (sem0, sem1)

        def fire(g, buf, sem):
            for c in range(GROUP):
                pltpu.async_copy(
                    table_hbm.at[idx_v.at[g * GROUP + c]],
                    buf.at[pl.ds(c * CHUNK, CHUNK)],
                    sem,
                )

        def drain(g, buf, sem):
            pltpu.make_async_copy(
                out_hbm.at[pl.ds(base + g * ROWS, ROWS)], buf, sem
            ).wait()

        pltpu.sync_copy(idx_hbm.at[pl.ds(wid * idx_rows_w, idx_rows_w)], idx_v)

        fire(0, buf0, sem0)

        def outer(i, carry):
            og = i * NBUF
            for b in range(NBUF):
                g = og + b
                nb = 1 - b

                @pl.when(g + 1 < n_groups)
                def _():
                    fire(g + 1, bufs[nb], sems[nb])

                drain(g, bufs[b], sems[b])
                pltpu.sync_copy(bufs[b], out_hbm.at[pl.ds(base + g * ROWS, ROWS)])
            return carry

        lax.fori_loop(0, n_groups // NBUF, outer, 0)

    return lookup


def kernel(x, table):
    b, l = x.shape
    emb = table.shape[1]
    n_total = b * l
    xf = x.reshape(n_total // CHUNK, CHUNK)
    out = _build_lookup(n_total, emb)(xf, table)
    return out.reshape(b, l, emb)
